# Initial kernel scaffold; baseline (speedup 1.0000x reference)
import jax
import jax.numpy as jnp
from jax.experimental import pallas as pl


def kernel(table, idx, targets):
    raise NotImplementedError("write your pallas kernel here")


# speedup vs baseline: 1.6109x; 1.6109x over previous
import functools

import jax
import jax.numpy as jnp
from jax import lax
from jax.experimental import pallas as pl
from jax.experimental.pallas import tpu as pltpu
from jax.experimental.pallas import tpu_sc as plsc

VOCAB = 1000
NTOK = 1024 * 50
NC, NS, L = 2, 16, 16
NW = NC * NS
RPT = NTOK // NW
CH = 32
NCHUNK = RPT // CH


def _lse_body(table_ref, out_ref):
    x = table_ref[...]
    m = jnp.max(x, axis=1, keepdims=True)
    s = jnp.sum(jnp.exp(x - m), axis=1, keepdims=True)
    out_ref[...] = m + jnp.log(s)


def _loss_body(part_ref, out_ref):
    out_ref[...] = (jnp.sum(part_ref[...]) / NTOK).reshape(1, 1)


def _sc_body(table_hbm, idx_hbm, tgt_hbm, lse_hbm, out_hbm, part_hbm,
             idx_v, tgt_v, lse_v, buf, acc_v, gsem):
    wid = lax.axis_index("s") * NC + lax.axis_index("c")
    base = wid * RPT
    pltpu.sync_copy(idx_hbm.at[pl.ds(base, RPT)], idx_v)
    pltpu.sync_copy(tgt_hbm.at[pl.ds(base, RPT)], tgt_v)
    pltpu.sync_copy(lse_hbm, lse_v)

    def chunk(g, acc):
        off = g * CH
        pltpu.async_copy(table_hbm.at[idx_v.at[pl.ds(off, CH)]], buf,
                         gsem).wait()
        for j in range(CH // L):
            lo = off + j * L
            rid = lax.iota(jnp.int32, L) + j * L
            ivec = idx_v[pl.ds(lo, L)]
            tvec = tgt_v[pl.ds(lo, L)]
            lse_g = plsc.load_gather(lse_v, [ivec])
            pick = plsc.load_gather(buf, [rid, tvec])
            acc = acc + (lse_g - pick)
        pltpu.sync_copy(buf, out_hbm.at[pl.ds(base + off, CH)])
        return acc

    acc = lax.fori_loop(0, NCHUNK, chunk, jnp.zeros((L,), jnp.float32))
    acc_v[...] = acc
    pltpu.sync_copy(acc_v, part_hbm.at[wid])


_sc_gather = functools.partial(
    pl.kernel,
    out_type=[jax.ShapeDtypeStruct((NTOK, VOCAB), jnp.float32),
              jax.ShapeDtypeStruct((NW, L), jnp.float32)],
    mesh=plsc.VectorSubcoreMesh(core_axis_name="c", subcore_axis_name="s"),
    compiler_params=pltpu.CompilerParams(needs_layout_passes=False,
                                         use_tc_tiling_on_sc=False),
    scratch_types=[
        pltpu.VMEM((RPT,), jnp.int32),
        pltpu.VMEM((RPT,), jnp.int32),
        pltpu.VMEM((VOCAB,), jnp.float32),
        pltpu.VMEM((CH, VOCAB), jnp.float32),
        pltpu.VMEM((L,), jnp.float32),
        pltpu.SemaphoreType.DMA,
    ],
)(_sc_body)


def kernel(table, idx, targets):
    idx_flat = idx.reshape(-1)
    tgt_flat = targets.reshape(-1)
    lse = pl.pallas_call(
        _lse_body,
        out_shape=jax.ShapeDtypeStruct((VOCAB, 1), jnp.float32),
    )(table).reshape(-1)
    logits, partials = _sc_gather(table, idx_flat, tgt_flat, lse)
    loss = pl.pallas_call(
        _loss_body,
        out_shape=jax.ShapeDtypeStruct((1, 1), jnp.float32),
    )(partials)[0, 0]
    return logits, loss


# speedup vs baseline: 1.6882x; 1.0480x over previous
import functools

import jax
import jax.numpy as jnp
from jax import lax
from jax.experimental import pallas as pl
from jax.experimental.pallas import tpu as pltpu
from jax.experimental.pallas import tpu_sc as plsc

VOCAB = 1000
NTOK = 1024 * 50
NC, NS, L = 2, 16, 16
NW = NC * NS
RPT = NTOK // NW
CH = 32
NCHUNK = RPT // CH


def _lse_body(table_ref, out_ref):
    x = table_ref[...]
    m = jnp.max(x, axis=1, keepdims=True)
    s = jnp.sum(jnp.exp(x - m), axis=1, keepdims=True)
    out_ref[...] = m + jnp.log(s)


def _loss_body(part_ref, out_ref):
    out_ref[...] = (jnp.sum(part_ref[...]) / NTOK).reshape(1, 1)


NPAIR = NCHUNK // 2


def _sc_body(table_hbm, idx_hbm, tgt_hbm, lse_hbm, out_hbm, part_hbm,
             idx_v, tgt_v, lse_v, buf0, buf1, acc_v,
             gsem0, gsem1, ssem0, ssem1):
    wid = lax.axis_index("s") * NC + lax.axis_index("c")
    base = wid * RPT
    pltpu.sync_copy(idx_hbm.at[pl.ds(base, RPT)], idx_v)
    pltpu.sync_copy(tgt_hbm.at[pl.ds(base, RPT)], tgt_v)
    pltpu.sync_copy(lse_hbm, lse_v)

    def gather(c, buf, sem):
        return pltpu.make_async_copy(
            table_hbm.at[idx_v.at[pl.ds(c * CH, CH)]], buf, sem)

    def scatter(c, buf, sem):
        return pltpu.make_async_copy(
            buf, out_hbm.at[pl.ds(base + c * CH, CH)], sem)

    def loss(buf, c, acc):
        for j in range(CH // L):
            lo = c * CH + j * L
            rid = lax.iota(jnp.int32, L) + j * L
            ivec = idx_v[pl.ds(lo, L)]
            tvec = tgt_v[pl.ds(lo, L)]
            acc = acc + (plsc.load_gather(lse_v, [ivec])
                         - plsc.load_gather(buf, [rid, tvec]))
        return acc

    gather(0, buf0, gsem0).start()

    def step(k, acc):
        a = 2 * k
        b = a + 1
        gather(a, buf0, gsem0).wait()

        @pl.when(k > 0)
        def _():
            scatter(b - 2, buf1, ssem1).wait()

        gather(b, buf1, gsem1).start()
        acc = loss(buf0, a, acc)
        scatter(a, buf0, ssem0).start()
        gather(b, buf1, gsem1).wait()

        @pl.when(k < NPAIR - 1)
        def _():
            scatter(a, buf0, ssem0).wait()
            gather(a + 2, buf0, gsem0).start()

        acc = loss(buf1, b, acc)
        scatter(b, buf1, ssem1).start()
        return acc

    acc = lax.fori_loop(0, NPAIR, step, jnp.zeros((L,), jnp.float32))
    scatter(2 * NPAIR - 2, buf0, ssem0).wait()
    scatter(2 * NPAIR - 1, buf1, ssem1).wait()
    acc_v[...] = acc
    pltpu.sync_copy(acc_v, part_hbm.at[wid])


_sc_gather = functools.partial(
    pl.kernel,
    out_type=[jax.ShapeDtypeStruct((NTOK, VOCAB), jnp.float32),
              jax.ShapeDtypeStruct((NW, L), jnp.float32)],
    mesh=plsc.VectorSubcoreMesh(core_axis_name="c", subcore_axis_name="s"),
    compiler_params=pltpu.CompilerParams(needs_layout_passes=False,
                                         use_tc_tiling_on_sc=False),
    scratch_types=[
        pltpu.VMEM((RPT,), jnp.int32),
        pltpu.VMEM((RPT,), jnp.int32),
        pltpu.VMEM((VOCAB,), jnp.float32),
        pltpu.VMEM((CH, VOCAB), jnp.float32),
        pltpu.VMEM((CH, VOCAB), jnp.float32),
        pltpu.VMEM((L,), jnp.float32),
        pltpu.SemaphoreType.DMA,
        pltpu.SemaphoreType.DMA,
        pltpu.SemaphoreType.DMA,
        pltpu.SemaphoreType.DMA,
    ],
)(_sc_body)


def kernel(table, idx, targets):
    idx_flat = idx.reshape(-1)
    tgt_flat = targets.reshape(-1)
    lse = pl.pallas_call(
        _lse_body,
        out_shape=jax.ShapeDtypeStruct((VOCAB, 1), jnp.float32),
    )(table).reshape(-1)
    logits, partials = _sc_gather(table, idx_flat, tgt_flat, lse)
    loss = pl.pallas_call(
        _loss_body,
        out_shape=jax.ShapeDtypeStruct((1, 1), jnp.float32),
    )(partials)[0, 0]
    return logits, loss


# speedup vs baseline: 2.4525x; 1.4527x over previous
import functools

import jax
import jax.numpy as jnp
from jax import lax
from jax.experimental import pallas as pl
from jax.experimental.pallas import tpu as pltpu
from jax.experimental.pallas import tpu_sc as plsc

VOCAB = 1000
VPAD = 1024
NTILE = VPAD // 128
TAIL = VOCAB - 128 * (NTILE - 1)
NTOK = 1024 * 50
NC, NS, L = 2, 16, 16
NW = NC * NS
RPT = NTOK // NW
RPAD = 2048
NCHUNK = RPT // L
NP = NCHUNK // 2


def _lse_body(table_ref, out_ref):
    x = table_ref[...]
    m = jnp.max(x, axis=1, keepdims=True)
    s = jnp.sum(jnp.exp(x - m), axis=1, keepdims=True)
    lse = m + jnp.log(s)
    out_ref[...] = jnp.concatenate(
        [lse, jnp.zeros((VPAD - VOCAB, 1), jnp.float32)], axis=0)


def _loss_body(part_ref, out_ref):
    out_ref[...] = (jnp.sum(part_ref[...]) / NTOK).reshape(1, 1)


def _sc_body(table8_hbm, idx_hbm, tgt_hbm, lse_hbm, out_hbm, tails_hbm,
             part_hbm, idx_v, tgt_v, lse_v, iidx0, iidx1, buf0, buf1, acc_v,
             gsem0, gsem1, ssem0, ssem1):
    wid = lax.axis_index("s") * NC + lax.axis_index("c")
    rbase = wid * RPT
    pltpu.sync_copy(idx_hbm.at[pl.ds(wid * RPAD, RPAD)], idx_v)
    pltpu.sync_copy(tgt_hbm.at[pl.ds(wid * RPAD, RPAD)], tgt_v)
    pltpu.sync_copy(lse_hbm, lse_v)

    lane = lax.iota(jnp.int32, L)

    def prep(c, iidx):
        v = idx_v[pl.ds(c * L, L)]
        bse = ((v >> 3) << 6) + (v & 7)
        for t in range(NTILE):
            iidx[pl.ds(t * L, L)] = bse + t * 8

    def gather(buf, iidx, sem):
        return pltpu.make_async_copy(table8_hbm.at[iidx], buf, sem)

    def scatters(c, buf, sem):
        r0 = rbase + c * L
        cps = []
        for t in range(NTILE - 1):
            cps.append(pltpu.make_async_copy(
                buf.at[pl.ds(t * L, L)],
                out_hbm.at[pl.ds(r0, L), pl.ds(t * 128, 128)], sem))
        cps.append(pltpu.make_async_copy(
            buf.at[pl.ds((NTILE - 1) * L, L)],
            tails_hbm.at[pl.ds(r0, L)], sem))
        return cps


    def start_scatters(c, buf, sem):
        for cp in scatters(c, buf, sem):
            cp.start()

    def wait_scatters(c, buf, sem):
        for cp in scatters(c, buf, sem):
            cp.wait()

    def loss(c, buf, acc):
        v = idx_v[pl.ds(c * L, L)]
        tg = tgt_v[pl.ds(c * L, L)]
        trow = ((tg >> 7) << 4) + lane
        tcol = tg & 127
        pick = plsc.load_gather(buf, [trow, tcol])
        lseg = plsc.load_gather(lse_v, [v])
        return acc + (lseg - pick)

    prep(0, iidx0)
    gather(buf0, iidx0, gsem0).start()

    def step(k, acc):
        a = 2 * k
        b = a + 1
        gather(buf0, iidx0, gsem0).wait()
        prep(b, iidx1)

        @pl.when(k > 0)
        def _():
            wait_scatters(b - 2, buf1, ssem1)

        gather(buf1, iidx1, gsem1).start()
        acc = loss(a, buf0, acc)
        start_scatters(a, buf0, ssem0)
        gather(buf1, iidx1, gsem1).wait()

        @pl.when(k < NP - 1)
        def _():
            wait_scatters(a, buf0, ssem0)
            prep(a + 2, iidx0)
            gather(buf0, iidx0, gsem0).start()

        acc = loss(b, buf1, acc)
        start_scatters(b, buf1, ssem1)
        return acc

    acc = lax.fori_loop(0, NP, step, jnp.zeros((L,), jnp.float32))
    wait_scatters(2 * NP - 2, buf0, ssem0)
    wait_scatters(2 * NP - 1, buf1, ssem1)
    zero = jnp.zeros((L,), jnp.float32)
    for j in range(8):
        acc_v[pl.ds(j * L, L)] = acc if j == 0 else zero
    pltpu.sync_copy(acc_v, part_hbm.at[pl.ds(wid * 128, 128)])


_sc_gather = functools.partial(
    pl.kernel,
    out_type=[jax.ShapeDtypeStruct((NTOK, VOCAB), jnp.float32),
              jax.ShapeDtypeStruct((NTOK, 128), jnp.float32),
              jax.ShapeDtypeStruct((NW * 128,), jnp.float32)],
    mesh=plsc.VectorSubcoreMesh(core_axis_name="c", subcore_axis_name="s"),
    compiler_params=pltpu.CompilerParams(needs_layout_passes=False,
                                         use_tc_tiling_on_sc=True),
    scratch_types=[
        pltpu.VMEM((RPAD,), jnp.int32),
        pltpu.VMEM((RPAD,), jnp.int32),
        pltpu.VMEM((VPAD,), jnp.float32),
        pltpu.VMEM((128,), jnp.int32),
        pltpu.VMEM((128,), jnp.int32),
        pltpu.VMEM((128, 128), jnp.float32),
        pltpu.VMEM((128, 128), jnp.float32),
        pltpu.VMEM((128,), jnp.float32),
        pltpu.SemaphoreType.DMA,
        pltpu.SemaphoreType.DMA,
        pltpu.SemaphoreType.DMA,
        pltpu.SemaphoreType.DMA,
    ],
)(_sc_body)


def kernel(table, idx, targets):
    table_p = jnp.pad(table, ((0, 0), (0, VPAD - VOCAB)))
    table8 = (table_p.reshape(VOCAB // 8, 8, NTILE, 128)
              .transpose(0, 2, 1, 3).reshape(VOCAB // 8 * NTILE * 8, 128))
    idx_pad = jnp.pad(idx.reshape(NW, RPT), ((0, 0), (0, RPAD - RPT)))
    tgt_pad = jnp.pad(targets.reshape(NW, RPT), ((0, 0), (0, RPAD - RPT)))
    lse = pl.pallas_call(
        _lse_body,
        out_shape=jax.ShapeDtypeStruct((VPAD, 1), jnp.float32),
    )(table).reshape(-1)
    logits0, tails, partials = _sc_gather(table8, idx_pad.reshape(-1),
                                          tgt_pad.reshape(-1), lse)
    logits = lax.dynamic_update_slice(logits0, tails[:, :TAIL],
                                      (0, 128 * (NTILE - 1)))
    loss = pl.pallas_call(
        _loss_body,
        out_shape=jax.ShapeDtypeStruct((1, 1), jnp.float32),
    )(partials.reshape(NW, 128))[0, 0]
    return logits, loss


# speedup vs baseline: 2.6601x; 1.0846x over previous
import functools

import jax
import jax.numpy as jnp
from jax import lax
from jax.experimental import pallas as pl
from jax.experimental.pallas import tpu as pltpu
from jax.experimental.pallas import tpu_sc as plsc

VOCAB = 1000
VPAD = 1024
NTILE = VPAD // 128
TAIL = VOCAB - 128 * (NTILE - 1)
NTOK = 1024 * 50
NC, NS, L = 2, 16, 16
NW = NC * NS
RPT = NTOK // NW
RPAD = 2048
CH = 32
NCHUNK = RPT // CH
NP = NCHUNK // 2


def _lse_body(table_ref, out_ref):
    x = table_ref[...]
    m = jnp.max(x, axis=1, keepdims=True)
    s = jnp.sum(jnp.exp(x - m), axis=1, keepdims=True)
    lse = m + jnp.log(s)
    out_ref[...] = jnp.concatenate(
        [lse, jnp.zeros((VPAD - VOCAB, 1), jnp.float32)], axis=0)


def _loss_body(part_ref, out_ref):
    out_ref[...] = (jnp.sum(part_ref[...]) / NTOK).reshape(1, 1)


def _sc_body(table_hbm, idx_hbm, tgt_hbm, lse_hbm, out_hbm, tails_hbm,
             part_hbm, idx_v, tgt_v, lse_v, buf0, buf1, acc_v,
             gsem0, gsem1, ssem0, ssem1):
    wid = lax.axis_index("s") * NC + lax.axis_index("c")
    rbase = wid * RPT
    pltpu.sync_copy(idx_hbm.at[pl.ds(wid * RPAD, RPAD)], idx_v)
    pltpu.sync_copy(tgt_hbm.at[pl.ds(wid * RPAD, RPAD)], tgt_v)
    pltpu.sync_copy(lse_hbm, lse_v)

    lane = lax.iota(jnp.int32, L)

    def gather(c, buf, sem):
        return pltpu.make_async_copy(
            table_hbm.at[idx_v.at[pl.ds(c * CH, CH)]], buf, sem)

    def scatters(c, buf, sem):
        r0 = rbase + c * CH
        cps = []
        for t in range(NTILE - 1):
            cps.append(pltpu.make_async_copy(
                buf.at[pl.ds(0, CH), pl.ds(t * 128, 128)],
                out_hbm.at[pl.ds(r0, CH), pl.ds(t * 128, 128)], sem))
        cps.append(pltpu.make_async_copy(
            buf.at[pl.ds(0, CH), pl.ds((NTILE - 1) * 128, 128)],
            tails_hbm.at[pl.ds(r0, CH)], sem))
        return cps

    def start_scatters(c, buf, sem):
        for cp in scatters(c, buf, sem):
            cp.start()

    def wait_scatters(c, buf, sem):
        for cp in scatters(c, buf, sem):
            cp.wait()

    def loss(c, buf, acc):
        for j in range(CH // L):
            lo = c * CH + j * L
            v = idx_v[pl.ds(lo, L)]
            tg = tgt_v[pl.ds(lo, L)]
            trow = lane + j * L
            pick = plsc.load_gather(buf, [trow, tg])
            lseg = plsc.load_gather(lse_v, [v])
            acc = acc + (lseg - pick)
        return acc

    gather(0, buf0, gsem0).start()

    def step(k, acc):
        a = 2 * k
        b = a + 1
        gather(a, buf0, gsem0).wait()

        @pl.when(k > 0)
        def _():
            wait_scatters(b - 2, buf1, ssem1)

        gather(b, buf1, gsem1).start()
        acc = loss(a, buf0, acc)
        start_scatters(a, buf0, ssem0)
        gather(b, buf1, gsem1).wait()

        @pl.when(k < NP - 1)
        def _():
            wait_scatters(a, buf0, ssem0)
            gather(a + 2, buf0, gsem0).start()

        acc = loss(b, buf1, acc)
        start_scatters(b, buf1, ssem1)
        return acc

    acc = lax.fori_loop(0, NP, step, jnp.zeros((L,), jnp.float32))
    wait_scatters(2 * NP - 2, buf0, ssem0)
    wait_scatters(2 * NP - 1, buf1, ssem1)
    zero = jnp.zeros((L,), jnp.float32)
    for j in range(8):
        acc_v[pl.ds(j * L, L)] = acc if j == 0 else zero
    pltpu.sync_copy(acc_v, part_hbm.at[pl.ds(wid * 128, 128)])


_sc_gather = functools.partial(
    pl.kernel,
    out_type=[jax.ShapeDtypeStruct((NTOK, VOCAB), jnp.float32),
              jax.ShapeDtypeStruct((NTOK, 128), jnp.float32),
              jax.ShapeDtypeStruct((NW * 128,), jnp.float32)],
    mesh=plsc.VectorSubcoreMesh(core_axis_name="c", subcore_axis_name="s"),
    compiler_params=pltpu.CompilerParams(needs_layout_passes=False,
                                         use_tc_tiling_on_sc=True),
    scratch_types=[
        pltpu.VMEM((RPAD,), jnp.int32),
        pltpu.VMEM((RPAD,), jnp.int32),
        pltpu.VMEM((VPAD,), jnp.float32),
        pltpu.VMEM((CH, VPAD), jnp.float32),
        pltpu.VMEM((CH, VPAD), jnp.float32),
        pltpu.VMEM((128,), jnp.float32),
        pltpu.SemaphoreType.DMA,
        pltpu.SemaphoreType.DMA,
        pltpu.SemaphoreType.DMA,
        pltpu.SemaphoreType.DMA,
    ],
)(_sc_body)


def kernel(table, idx, targets):
    table_p = jnp.pad(table, ((0, 0), (0, VPAD - VOCAB)))
    idx_pad = jnp.pad(idx.reshape(NW, RPT), ((0, 0), (0, RPAD - RPT)))
    tgt_pad = jnp.pad(targets.reshape(NW, RPT), ((0, 0), (0, RPAD - RPT)))
    lse = pl.pallas_call(
        _lse_body,
        out_shape=jax.ShapeDtypeStruct((VPAD, 1), jnp.float32),
    )(table).reshape(-1)
    logits0, tails, partials = _sc_gather(table_p, idx_pad.reshape(-1),
                                          tgt_pad.reshape(-1), lse)
    logits = lax.dynamic_update_slice(logits0, tails[:, :TAIL],
                                      (0, 128 * (NTILE - 1)))
    loss = pl.pallas_call(
        _loss_body,
        out_shape=jax.ShapeDtypeStruct((1, 1), jnp.float32),
    )(partials.reshape(NW, 128))[0, 0]
    return logits, loss
